# Initial kernel scaffold; baseline (speedup 1.0000x reference)
import functools

import jax
import jax.numpy as jnp
from jax import lax
from jax.experimental import pallas as pl
from jax.experimental.pallas import tpu as pltpu
from jax.experimental.pallas import tpu_sc as plsc

N = 10000
D = 128
E = 320000
NW = 32
EW = E // NW
CH_A = 1000
CH_C = 400
RB = 2000
NPT = N // 16
NEG = -3.0e38

_mesh = plsc.VectorSubcoreMesh(core_axis_name="c", subcore_axis_name="s")


@functools.partial(
    pl.kernel,
    out_type=[
        jax.ShapeDtypeStruct((E,), jnp.float32),
        jax.ShapeDtypeStruct((NW, N), jnp.float32),
    ],
    mesh=_mesh,
    scratch_types=[
        pltpu.VMEM((N,), jnp.float32),
        pltpu.VMEM((N,), jnp.float32),
        pltpu.VMEM((N,), jnp.float32),
        pltpu.VMEM((CH_A,), jnp.int32),
        pltpu.VMEM((CH_A,), jnp.int32),
        pltpu.VMEM((CH_A,), jnp.float32),
    ],
)
def _sc_alpha_max(src_hbm, dst_hbm, asrc_hbm, adst_hbm,
                  alpha_hbm, amax_hbm,
                  asrc_v, adst_v, amax_v, src_v, dst_v, al_v):
    wid = lax.axis_index("s") * 2 + lax.axis_index("c")
    pltpu.sync_copy(asrc_hbm, asrc_v)
    pltpu.sync_copy(adst_hbm, adst_v)

    neg = jnp.full((16,), NEG, jnp.float32)

    @pl.loop(0, N, step=16)
    def _(i):
        amax_v[pl.ds(i, 16)] = neg

    base = wid * EW

    @pl.loop(0, EW, step=CH_A)
    def _(c0):
        pltpu.sync_copy(src_hbm.at[pl.ds(base + c0, CH_A)], src_v)
        pltpu.sync_copy(dst_hbm.at[pl.ds(base + c0, CH_A)], dst_v)

        @pl.loop(0, CH_A, step=16)
        def _(i):
            s16 = src_v[pl.ds(i, 16)]
            d16 = dst_v[pl.ds(i, 16)]
            a = plsc.load_gather(asrc_v, [s16]) + plsc.load_gather(adst_v, [d16])
            a = jnp.where(a >= 0.0, a, 0.2 * a)
            al_v[pl.ds(i, 16)] = a

            def _cond(m):
                return jnp.any(m)

            def _body(m):
                cur = plsc.load_gather(amax_v, [d16])
                plsc.store_scatter(amax_v, [d16], jnp.maximum(cur, a), mask=m)
                cur2 = plsc.load_gather(amax_v, [d16])
                return jnp.logical_and(m, cur2 < a)

            lax.while_loop(_cond, _body, jnp.ones((16,), jnp.bool_))

        pltpu.sync_copy(al_v, alpha_hbm.at[pl.ds(base + c0, CH_A)])

    pltpu.sync_copy(amax_v, amax_hbm.at[wid])


@functools.partial(
    pl.kernel,
    out_type=[
        jax.ShapeDtypeStruct((2 * N, D), jnp.float32),
        jax.ShapeDtypeStruct((NW, N), jnp.float32),
    ],
    mesh=_mesh,
    scratch_types=[
        pltpu.VMEM((N,), jnp.float32),
        pltpu.VMEM((N,), jnp.float32),
        pltpu.VMEM((CH_C,), jnp.int32),
        pltpu.VMEM((CH_C,), jnp.int32),
        pltpu.VMEM((CH_C,), jnp.float32),
        pltpu.VMEM((CH_C, D), jnp.float32),
        pltpu.VMEM_SHARED((N, D), jnp.float32),
        pltpu.SemaphoreType.DMA,
    ],
)
def _sc_rows(src_hbm, dst_hbm, alpha_hbm, amax_hbm, h_hbm,
             racc_hbm, dpart_hbm,
             amax_v, den_v, src_v, dst_v, w_v, rows_v, acc_sh, sem):
    cid = lax.axis_index("c")
    sid = lax.axis_index("s")
    wid = sid * 2 + cid

    z16 = jnp.zeros((16,), jnp.float32)

    @pl.loop(0, CH_C)
    def _(r):
        for c in range(8):
            rows_v[r, pl.ds(c * 16, 16)] = z16

    row0 = sid * NPT
    pltpu.sync_copy(rows_v, acc_sh.at[pl.ds(row0, CH_C)])
    pltpu.sync_copy(rows_v.at[pl.ds(0, NPT - CH_C)],
                    acc_sh.at[pl.ds(row0 + CH_C, NPT - CH_C)])

    pltpu.sync_copy(amax_hbm, amax_v)

    @pl.loop(0, N, step=16)
    def _(i):
        den_v[pl.ds(i, 16)] = z16

    plsc.subcore_barrier()

    base = wid * EW

    @pl.loop(0, EW, step=CH_C)
    def _(c0):
        pltpu.sync_copy(src_hbm.at[pl.ds(base + c0, CH_C)], src_v)
        pltpu.sync_copy(dst_hbm.at[pl.ds(base + c0, CH_C)], dst_v)
        pltpu.sync_copy(alpha_hbm.at[pl.ds(base + c0, CH_C)], w_v)

        pltpu.async_copy(h_hbm.at[src_v], rows_v, sem).wait()

        @pl.loop(0, CH_C, step=16)
        def _(i):
            d16 = dst_v[pl.ds(i, 16)]
            w = jnp.exp(w_v[pl.ds(i, 16)] - plsc.load_gather(amax_v, [d16]))
            w_v[pl.ds(i, 16)] = w
            plsc.addupdate_scatter(den_v, [d16], w)

        @pl.loop(0, CH_C)
        def _(r):
            ws = plsc.load_gather(w_v, [lax.broadcast(r, (16,))])
            for c in range(8):
                sl = (r, pl.ds(c * 16, 16))
                rows_v[sl] = rows_v[sl] * ws

        pltpu.sync_copy(rows_v, acc_sh.at[dst_v], add=True)

    pltpu.sync_copy(den_v, dpart_hbm.at[wid])

    plsc.subcore_barrier()
    pltpu.sync_copy(acc_sh.at[pl.ds(row0, CH_C)],
                    racc_hbm.at[pl.ds(cid * N + row0, CH_C)])
    pltpu.sync_copy(acc_sh.at[pl.ds(row0 + CH_C, NPT - CH_C)],
                    racc_hbm.at[pl.ds(cid * N + row0 + CH_C, NPT - CH_C)])


def _prep_body(x_ref, w_ref, av_ref, bv_ref, h_ref, asrc_ref, adst_ref, sal_ref):
    h = jnp.dot(x_ref[...], w_ref[...], preferred_element_type=jnp.float32)
    h_ref[...] = h
    s = jnp.sum(h * av_ref[...][None, :], axis=1)
    t = jnp.sum(h * bv_ref[...][None, :], axis=1)
    asrc_ref[...] = s
    adst_ref[...] = t
    u = s + t
    sal_ref[...] = jnp.where(u >= 0.0, u, 0.2 * u)


def _prep(x, W, a_src, a_dst):
    return pl.pallas_call(
        _prep_body,
        grid=(N // RB,),
        in_specs=[
            pl.BlockSpec((RB, D), lambda i: (i, 0)),
            pl.BlockSpec((D, D), lambda i: (0, 0)),
            pl.BlockSpec((D,), lambda i: (0,)),
            pl.BlockSpec((D,), lambda i: (0,)),
        ],
        out_specs=[
            pl.BlockSpec((RB, D), lambda i: (i, 0)),
            pl.BlockSpec((RB,), lambda i: (i,)),
            pl.BlockSpec((RB,), lambda i: (i,)),
            pl.BlockSpec((RB,), lambda i: (i,)),
        ],
        out_shape=[
            jax.ShapeDtypeStruct((N, D), jnp.float32),
            jax.ShapeDtypeStruct((N,), jnp.float32),
            jax.ShapeDtypeStruct((N,), jnp.float32),
            jax.ShapeDtypeStruct((N,), jnp.float32),
        ],
    )(x, W, a_src, a_dst)


def _comb_body(amaxp_ref, sal_ref, amax_ref, selfw_ref):
    m = jnp.maximum(jnp.max(amaxp_ref[...], axis=0), sal_ref[...])
    amax_ref[...] = m
    selfw_ref[...] = jnp.exp(sal_ref[...] - m)


def _comb(amaxp, sal):
    return pl.pallas_call(
        _comb_body,
        grid=(N // RB,),
        in_specs=[
            pl.BlockSpec((NW, RB), lambda i: (0, i)),
            pl.BlockSpec((RB,), lambda i: (i,)),
        ],
        out_specs=[
            pl.BlockSpec((RB,), lambda i: (i,)),
            pl.BlockSpec((RB,), lambda i: (i,)),
        ],
        out_shape=[
            jax.ShapeDtypeStruct((N,), jnp.float32),
            jax.ShapeDtypeStruct((N,), jnp.float32),
        ],
    )(amaxp, sal)


def _finish_body(apply_gelu, r0_ref, r1_ref, dp_ref, sw_ref, h_ref, b_ref, out_ref):
    sw = sw_ref[...]
    den = jnp.sum(dp_ref[...], axis=0) + sw
    num = r0_ref[...] + r1_ref[...] + sw[:, None] * h_ref[...]
    out = num / den[:, None] + b_ref[...][None, :]
    if apply_gelu:
        out = jax.nn.gelu(out, approximate=False)
    out_ref[...] = out


def _finish(racc, dpart, selfw, h, b, apply_gelu):
    nb = N // RB
    return pl.pallas_call(
        functools.partial(_finish_body, apply_gelu),
        grid=(nb,),
        in_specs=[
            pl.BlockSpec((RB, D), lambda i: (i, 0)),
            pl.BlockSpec((RB, D), lambda i: (i + nb, 0)),
            pl.BlockSpec((NW, RB), lambda i: (0, i)),
            pl.BlockSpec((RB,), lambda i: (i,)),
            pl.BlockSpec((RB, D), lambda i: (i, 0)),
            pl.BlockSpec((D,), lambda i: (0,)),
        ],
        out_specs=pl.BlockSpec((RB, D), lambda i: (i, 0)),
        out_shape=jax.ShapeDtypeStruct((N, D), jnp.float32),
    )(racc, racc, dpart, selfw, h, b)


def kernel(embeded_nodes_features, edges_connectivity,
           W0, a_src0, a_dst0, b0, W1, a_src1, a_dst1, b1):
    src = edges_connectivity[0].astype(jnp.int32)
    dst = edges_connectivity[1].astype(jnp.int32)

    def layer(xin, W, a_s, a_d, b, apply_gelu):
        h, asrc, adst, sal = _prep(xin, W, a_s, a_d)
        alpha, amaxp = _sc_alpha_max(src, dst, asrc, adst)
        amax, selfw = _comb(amaxp, sal)
        racc, dpart = _sc_rows(src, dst, alpha, amax, h)
        return _finish(racc, dpart, selfw, h, b, apply_gelu)

    x1 = layer(embeded_nodes_features, W0, a_src0, a_dst0, b0, True)
    return layer(x1, W1, a_src1, a_dst1, b1, False)


# speedup vs baseline: 12.7635x; 12.7635x over previous
import dataclasses
import functools

import jax
import jax.numpy as jnp
from jax import lax
from jax.experimental import pallas as pl
from jax.experimental.pallas import tpu as pltpu
from jax.experimental.pallas import tpu_sc as plsc

N = 10000
NP = 10240
D = 128
E = 320000
NW = 32
RB = 2048

_mesh = plsc.VectorSubcoreMesh(core_axis_name="c", subcore_axis_name="s")

_sc_params = pltpu.CompilerParams()
if "needs_layout_passes" in pltpu.CompilerParams.__dataclass_fields__:
    _sc_params = dataclasses.replace(_sc_params, needs_layout_passes=False)
_sc_params = dataclasses.replace(_sc_params, use_tc_tiling_on_sc=False)


QD = D // 4
ET = E // 16
CH = 2000
GW = 80
NWIN = CH // GW
NPT = NP // 16


@functools.partial(
    pl.kernel,
    out_type=[
        jax.ShapeDtypeStruct((4 * NP, QD), jnp.float32),
        jax.ShapeDtypeStruct((NW, NP), jnp.float32),
    ],
    mesh=_mesh,
    compiler_params=_sc_params,
    scratch_types=[
        pltpu.VMEM((NP,), jnp.float32),
        pltpu.VMEM((NP,), jnp.float32),
        pltpu.VMEM((NP,), jnp.float32),
        pltpu.VMEM((16,), jnp.float32),
        pltpu.VMEM((CH,), jnp.int32),
        pltpu.VMEM((CH,), jnp.int32),
        pltpu.VMEM((NWIN, GW), jnp.int32),
        pltpu.VMEM((CH,), jnp.int32),
        pltpu.VMEM((CH,), jnp.float32),
        pltpu.VMEM((CH, QD), jnp.float32),
        pltpu.VMEM_SHARED((NP, QD), jnp.float32),
        pltpu.SemaphoreType.DMA,
    ],
)
def _sc_edges(src_hbm, dst_hbm, asrc_hbm, adst_hbm, mb_hbm, h4_hbm,
              racc_hbm, dpart_hbm,
              asrc_v, adst_v, den_v, mb_v, src_v, dst_v, dst2_v, idx_v, w_v,
              rows_v, acc_sh, sem):
    cid = lax.axis_index("c")
    sid = lax.axis_index("s")
    wid = sid * 2 + cid

    pltpu.sync_copy(asrc_hbm, asrc_v)
    pltpu.sync_copy(adst_hbm, adst_v)
    pltpu.sync_copy(mb_hbm, mb_v)

    z16 = jnp.zeros((16,), jnp.float32)

    @pl.loop(0, NP, step=16)
    def _(i):
        den_v[pl.ds(i, 16)] = z16

    m16 = mb_v[pl.ds(0, 16)]
    base = sid * ET
    row0 = sid * NPT

    for p in range(2):
        q = cid * 2 + p

        @pl.loop(0, NPT)
        def _(r):
            for c in range(QD // 16):
                rows_v[r, pl.ds(c * 16, 16)] = z16

        pltpu.sync_copy(rows_v.at[pl.ds(0, NPT)], acc_sh.at[pl.ds(row0, NPT)])
        plsc.subcore_barrier()

        @pl.loop(0, ET, step=CH)
        def _(c0):
            pltpu.sync_copy(src_hbm.at[pl.ds(base + c0, CH)], src_v)
            pltpu.sync_copy(dst_hbm.at[pl.ds(base + c0, CH)], dst_v)

            @pl.loop(0, NWIN)
            def _(j):
                @pl.loop(0, GW, step=16)
                def _(t):
                    i = j * GW + t
                    s16 = src_v[pl.ds(i, 16)]
                    d16 = dst_v[pl.ds(i, 16)]
                    a = (plsc.load_gather(asrc_v, [s16])
                         + plsc.load_gather(adst_v, [d16]))
                    a = jnp.where(a >= 0.0, a, 0.2 * a)
                    w = jnp.exp(a - m16)
                    w_v[pl.ds(i, 16)] = w
                    plsc.addupdate_scatter(den_v, [d16], w)
                    idx_v[pl.ds(i, 16)] = s16 * 4 + q
                    dst2_v[j, pl.ds(t, 16)] = d16

            @pl.loop(0, NWIN)
            def _(j):
                pltpu.async_copy(
                    h4_hbm.at[idx_v.at[pl.ds(j * GW, GW)]],
                    rows_v.at[pl.ds(j * GW, GW)], sem).wait()

            @pl.loop(0, CH)
            def _(r):
                ws = plsc.load_gather(w_v, [lax.broadcast(r, (16,))])
                for c in range(QD // 16):
                    sl = (r, pl.ds(c * 16, 16))
                    rows_v[sl] = rows_v[sl] * ws

            @pl.loop(0, NWIN)
            def _(j):
                pltpu.sync_copy(rows_v.at[pl.ds(j * GW, GW)],
                                acc_sh.at[dst2_v.at[j]], add=True)

        plsc.subcore_barrier()
        pltpu.sync_copy(acc_sh.at[pl.ds(row0, NPT)],
                        racc_hbm.at[pl.ds(q * NP + row0, NPT)])
        plsc.subcore_barrier()

    pltpu.sync_copy(den_v, dpart_hbm.at[wid])


def _prep_body(x_ref, w_ref, av_ref, bv_ref,
               h_ref, asrc_ref, adst_ref, mb_ref, selfw_ref):
    h = jnp.dot(x_ref[...], w_ref[...], preferred_element_type=jnp.float32)
    h_ref[...] = h
    s = jnp.sum(h * av_ref[...][None, :], axis=1)
    t = jnp.sum(h * bv_ref[...][None, :], axis=1)
    asrc_ref[...] = s
    adst_ref[...] = t
    u = s + t
    sal = jnp.where(u >= 0.0, u, 0.2 * u)
    mraw = jnp.max(s) + jnp.max(t)
    m = jnp.where(mraw >= 0.0, mraw, 0.2 * mraw)
    mb_ref[...] = jnp.full((16,), m, jnp.float32)
    selfw_ref[...] = jnp.exp(sal - m)


def _prep(x, W, a_src, a_dst):
    return pl.pallas_call(
        _prep_body,
        grid=(1,),
        in_specs=[
            pl.BlockSpec((NP, D), lambda i: (0, 0)),
            pl.BlockSpec((D, D), lambda i: (0, 0)),
            pl.BlockSpec((D,), lambda i: (0,)),
            pl.BlockSpec((D,), lambda i: (0,)),
        ],
        out_specs=[
            pl.BlockSpec((NP, D), lambda i: (0, 0)),
            pl.BlockSpec((NP,), lambda i: (0,)),
            pl.BlockSpec((NP,), lambda i: (0,)),
            pl.BlockSpec((16,), lambda i: (0,)),
            pl.BlockSpec((NP,), lambda i: (0,)),
        ],
        out_shape=[
            jax.ShapeDtypeStruct((NP, D), jnp.float32),
            jax.ShapeDtypeStruct((NP,), jnp.float32),
            jax.ShapeDtypeStruct((NP,), jnp.float32),
            jax.ShapeDtypeStruct((16,), jnp.float32),
            jax.ShapeDtypeStruct((NP,), jnp.float32),
        ],
    )(x, W, a_src, a_dst)


def _finish_body(apply_gelu, r0_ref, r1_ref, r2_ref, r3_ref, dp_ref, sw_ref,
                 h_ref, b_ref, out_ref):
    sw = sw_ref[...]
    den = 0.25 * jnp.sum(dp_ref[...], axis=0) + sw
    acc = jnp.concatenate(
        [r0_ref[...], r1_ref[...], r2_ref[...], r3_ref[...]], axis=1)
    num = acc + sw[:, None] * h_ref[...]
    out = num / den[:, None] + b_ref[...][None, :]
    if apply_gelu:
        out = 0.5 * out * (1.0 + lax.erf(out * (2.0 ** -0.5)))
    out_ref[...] = out


def _finish(racc, dpart, selfw, h, b, apply_gelu):
    nb = NP // RB

    def qspec(q):
        return pl.BlockSpec((RB, QD), lambda i, q=q: (i + q * nb, 0))

    return pl.pallas_call(
        functools.partial(_finish_body, apply_gelu),
        grid=(nb,),
        in_specs=[
            qspec(0), qspec(1), qspec(2), qspec(3),
            pl.BlockSpec((NW, RB), lambda i: (0, i)),
            pl.BlockSpec((RB,), lambda i: (i,)),
            pl.BlockSpec((RB, D), lambda i: (i, 0)),
            pl.BlockSpec((D,), lambda i: (0,)),
        ],
        out_specs=pl.BlockSpec((RB, D), lambda i: (i, 0)),
        out_shape=jax.ShapeDtypeStruct((NP, D), jnp.float32),
    )(racc, racc, racc, racc, dpart, selfw, h, b)


def kernel(embeded_nodes_features, edges_connectivity,
           W0, a_src0, a_dst0, b0, W1, a_src1, a_dst1, b1):
    src = edges_connectivity[0].astype(jnp.int32)
    dst = edges_connectivity[1].astype(jnp.int32)

    def layer(xin, W, a_s, a_d, b, apply_gelu):
        h, asrc, adst, mb, selfw = _prep(xin, W, a_s, a_d)
        h4 = jnp.reshape(h, (4 * NP, QD))
        racc, dpart = _sc_edges(src, dst, asrc, adst, mb, h4)
        return _finish(racc, dpart, selfw, h, b, apply_gelu)

    xp = jnp.zeros((NP, D), jnp.float32).at[:N].set(embeded_nodes_features)
    x1 = layer(xp, W0, a_src0, a_dst0, b0, True)
    return layer(x1, W1, a_src1, a_dst1, b1, False)[:N]


# speedup vs baseline: 21.4901x; 1.6837x over previous
import dataclasses
import functools

import jax
import jax.numpy as jnp
from jax import lax
from jax.experimental import pallas as pl
from jax.experimental.pallas import tpu as pltpu
from jax.experimental.pallas import tpu_sc as plsc

N = 10000
NP = 10240
D = 128
E = 320000
NW = 32
RB = 2048

_mesh = plsc.VectorSubcoreMesh(core_axis_name="c", subcore_axis_name="s")

_sc_params = pltpu.CompilerParams()
if "needs_layout_passes" in pltpu.CompilerParams.__dataclass_fields__:
    _sc_params = dataclasses.replace(_sc_params, needs_layout_passes=False)
_sc_params = dataclasses.replace(_sc_params, use_tc_tiling_on_sc=False)


QD = D // 4
ET = E // 16
CH = 2000
GW = 80
NWIN = CH // GW
NPT = NP // 16


@functools.partial(
    pl.kernel,
    out_type=[
        jax.ShapeDtypeStruct((4 * NP, QD), jnp.float32),
        jax.ShapeDtypeStruct((NW, NP), jnp.float32),
    ],
    mesh=_mesh,
    compiler_params=_sc_params,
    scratch_types=[
        pltpu.VMEM((NP,), jnp.float32),
        pltpu.VMEM((NP,), jnp.float32),
        pltpu.VMEM((NP,), jnp.float32),
        pltpu.VMEM((16,), jnp.float32),
        pltpu.VMEM((CH,), jnp.int32),
        pltpu.VMEM((CH,), jnp.int32),
        pltpu.VMEM((NWIN, GW), jnp.int32),
        pltpu.VMEM((CH,), jnp.int32),
        pltpu.VMEM((CH,), jnp.float32),
        pltpu.VMEM((CH, QD), jnp.float32),
        pltpu.VMEM_SHARED((NP, QD), jnp.float32),
        pltpu.SemaphoreType.DMA,
        pltpu.SemaphoreType.DMA,
    ],
)
def _sc_edges(src_hbm, dst_hbm, asrc_hbm, adst_hbm, mb_hbm, h4_hbm,
              racc_hbm, dpart_hbm,
              asrc_v, adst_v, den_v, mb_v, src_v, dst_v, dst2_v, idx_v, w_v,
              rows_v, acc_sh, sem, sem2):
    cid = lax.axis_index("c")
    sid = lax.axis_index("s")
    wid = sid * 2 + cid

    pltpu.sync_copy(asrc_hbm, asrc_v)
    pltpu.sync_copy(adst_hbm, adst_v)
    pltpu.sync_copy(mb_hbm, mb_v)

    z16 = jnp.zeros((16,), jnp.float32)

    @pl.loop(0, NP, step=16)
    def _(i):
        den_v[pl.ds(i, 16)] = z16

    m16 = mb_v[pl.ds(0, 16)]
    base = sid * ET
    row0 = sid * NPT

    for p in range(2):
        q = cid * 2 + p

        @pl.loop(0, NPT)
        def _(r):
            for c in range(QD // 16):
                rows_v[r, pl.ds(c * 16, 16)] = z16

        pltpu.sync_copy(rows_v.at[pl.ds(0, NPT)], acc_sh.at[pl.ds(row0, NPT)])
        plsc.subcore_barrier()

        @pl.loop(0, ET, step=CH)
        def _(c0):
            pltpu.sync_copy(src_hbm.at[pl.ds(base + c0, CH)], src_v)
            pltpu.sync_copy(dst_hbm.at[pl.ds(base + c0, CH)], dst_v)

            @pl.loop(0, NWIN)
            def _(j):
                @pl.loop(0, GW, step=16)
                def _(t):
                    i = j * GW + t
                    s16 = src_v[pl.ds(i, 16)]
                    d16 = dst_v[pl.ds(i, 16)]
                    a = (plsc.load_gather(asrc_v, [s16])
                         + plsc.load_gather(adst_v, [d16]))
                    a = jnp.where(a >= 0.0, a, 0.2 * a)
                    w = jnp.exp(a - m16)
                    w_v[pl.ds(i, 16)] = w
                    plsc.addupdate_scatter(den_v, [d16], w)
                    idx_v[pl.ds(i, 16)] = s16 * 4 + q
                    dst2_v[j, pl.ds(t, 16)] = d16

                pltpu.async_copy(
                    h4_hbm.at[idx_v.at[pl.ds(j * GW, GW)]],
                    rows_v.at[pl.ds(j * GW, GW)], sem)

            @pl.loop(0, NWIN)
            def _(j):
                pltpu.make_async_copy(
                    h4_hbm.at[idx_v.at[pl.ds(j * GW, GW)]],
                    rows_v.at[pl.ds(j * GW, GW)], sem).wait()

            @pl.loop(0, CH)
            def _(r):
                ws = plsc.load_gather(w_v, [lax.broadcast(r, (16,))])
                for c in range(QD // 16):
                    sl = (r, pl.ds(c * 16, 16))
                    rows_v[sl] = rows_v[sl] * ws

            @pl.loop(0, NWIN)
            def _(j):
                pltpu.async_copy(rows_v.at[pl.ds(j * GW, GW)],
                                 acc_sh.at[dst2_v.at[j]], sem2, add=True)

            @pl.loop(0, NWIN)
            def _(j):
                pltpu.make_async_copy(rows_v.at[pl.ds(j * GW, GW)],
                                      acc_sh.at[dst2_v.at[j]], sem2).wait()

        plsc.subcore_barrier()
        pltpu.sync_copy(acc_sh.at[pl.ds(row0, NPT)],
                        racc_hbm.at[pl.ds(q * NP + row0, NPT)])
        plsc.subcore_barrier()

    pltpu.sync_copy(den_v, dpart_hbm.at[wid])


def _prep_body(x_ref, w_ref, av_ref, bv_ref,
               h_ref, asrc_ref, adst_ref, mb_ref, selfw_ref):
    h = jnp.dot(x_ref[...], w_ref[...], preferred_element_type=jnp.float32)
    h_ref[...] = h
    s = jnp.sum(h * av_ref[...][None, :], axis=1)
    t = jnp.sum(h * bv_ref[...][None, :], axis=1)
    asrc_ref[...] = s
    adst_ref[...] = t
    u = s + t
    sal = jnp.where(u >= 0.0, u, 0.2 * u)
    mraw = jnp.max(s) + jnp.max(t)
    m = jnp.where(mraw >= 0.0, mraw, 0.2 * mraw)
    mb_ref[...] = jnp.full((16,), m, jnp.float32)
    selfw_ref[...] = jnp.exp(sal - m)


def _prep(x, W, a_src, a_dst):
    return pl.pallas_call(
        _prep_body,
        grid=(1,),
        in_specs=[
            pl.BlockSpec((NP, D), lambda i: (0, 0)),
            pl.BlockSpec((D, D), lambda i: (0, 0)),
            pl.BlockSpec((D,), lambda i: (0,)),
            pl.BlockSpec((D,), lambda i: (0,)),
        ],
        out_specs=[
            pl.BlockSpec((NP, D), lambda i: (0, 0)),
            pl.BlockSpec((NP,), lambda i: (0,)),
            pl.BlockSpec((NP,), lambda i: (0,)),
            pl.BlockSpec((16,), lambda i: (0,)),
            pl.BlockSpec((NP,), lambda i: (0,)),
        ],
        out_shape=[
            jax.ShapeDtypeStruct((NP, D), jnp.float32),
            jax.ShapeDtypeStruct((NP,), jnp.float32),
            jax.ShapeDtypeStruct((NP,), jnp.float32),
            jax.ShapeDtypeStruct((16,), jnp.float32),
            jax.ShapeDtypeStruct((NP,), jnp.float32),
        ],
    )(x, W, a_src, a_dst)


def _finish_body(apply_gelu, r0_ref, r1_ref, r2_ref, r3_ref, dp_ref, sw_ref,
                 h_ref, b_ref, out_ref):
    sw = sw_ref[...]
    den = 0.25 * jnp.sum(dp_ref[...], axis=0) + sw
    acc = jnp.concatenate(
        [r0_ref[...], r1_ref[...], r2_ref[...], r3_ref[...]], axis=1)
    num = acc + sw[:, None] * h_ref[...]
    out = num / den[:, None] + b_ref[...][None, :]
    if apply_gelu:
        out = 0.5 * out * (1.0 + lax.erf(out * (2.0 ** -0.5)))
    out_ref[...] = out


def _finish(racc, dpart, selfw, h, b, apply_gelu):
    nb = NP // RB

    def qspec(q):
        return pl.BlockSpec((RB, QD), lambda i, q=q: (i + q * nb, 0))

    return pl.pallas_call(
        functools.partial(_finish_body, apply_gelu),
        grid=(nb,),
        in_specs=[
            qspec(0), qspec(1), qspec(2), qspec(3),
            pl.BlockSpec((NW, RB), lambda i: (0, i)),
            pl.BlockSpec((RB,), lambda i: (i,)),
            pl.BlockSpec((RB, D), lambda i: (i, 0)),
            pl.BlockSpec((D,), lambda i: (0,)),
        ],
        out_specs=pl.BlockSpec((RB, D), lambda i: (i, 0)),
        out_shape=jax.ShapeDtypeStruct((NP, D), jnp.float32),
    )(racc, racc, racc, racc, dpart, selfw, h, b)


def kernel(embeded_nodes_features, edges_connectivity,
           W0, a_src0, a_dst0, b0, W1, a_src1, a_dst1, b1):
    src = edges_connectivity[0].astype(jnp.int32)
    dst = edges_connectivity[1].astype(jnp.int32)

    def layer(xin, W, a_s, a_d, b, apply_gelu):
        h, asrc, adst, mb, selfw = _prep(xin, W, a_s, a_d)
        h4 = jnp.reshape(h, (4 * NP, QD))
        racc, dpart = _sc_edges(src, dst, asrc, adst, mb, h4)
        return _finish(racc, dpart, selfw, h, b, apply_gelu)

    xp = jnp.zeros((NP, D), jnp.float32).at[:N].set(embeded_nodes_features)
    x1 = layer(xp, W0, a_src0, a_dst0, b0, True)
    return layer(x1, W1, a_src1, a_dst1, b1, False)[:N]
